# Initial kernel scaffold; baseline (speedup 1.0000x reference)
import functools

import jax
import jax.numpy as jnp
from jax import lax
from jax.experimental import pallas as pl
from jax.experimental.pallas import tpu as pltpu
from jax.experimental.pallas import tpu_sc as plsc

B = 16384
D = 32
L = 20
NC = 2
NS = 16
NW = NC * NS
BPW = B // NW
CH = 128
NCH = BPW // CH


def _sc_gather(rec_g, ing_g, id_table, ing_table):
  mesh = plsc.VectorSubcoreMesh(
      core_axis_name="c", subcore_axis_name="s",
      num_cores=NC, num_subcores=NS)

  @functools.partial(
      pl.kernel,
      out_type=(jax.ShapeDtypeStruct((B, D), jnp.float32),
                jax.ShapeDtypeStruct((B, D), jnp.float32)),
      mesh=mesh,
      scratch_types=[
          pltpu.VMEM((NCH, CH), jnp.int32),
          pltpu.VMEM((L, NCH, CH), jnp.int32),
          pltpu.VMEM((BPW, D), jnp.float32),
          pltpu.VMEM((BPW, D), jnp.float32),
          pltpu.VMEM((BPW, D), jnp.float32),
          pltpu.SemaphoreType.DMA,
      ],
  )
  def k(rec_hbm, ing_hbm, idtab_hbm, ingtab_hbm, ido_hbm, ingo_hbm,
        ridx, iidx, idrows, acc, buf, sem):
    wid = lax.axis_index("s") * NC + lax.axis_index("c")
    base = wid * BPW
    pltpu.sync_copy(rec_hbm.at[wid], ridx)
    pltpu.sync_copy(ing_hbm.at[wid], iidx)

    cps = [pltpu.async_copy(idtab_hbm.at[ridx.at[kk]],
                            idrows.at[pl.ds(kk * CH, CH)], sem)
           for kk in range(NCH)]
    for c in cps:
      c.wait()
    pltpu.sync_copy(idrows, ido_hbm.at[pl.ds(base, BPW)])

    cps = [pltpu.async_copy(ingtab_hbm.at[iidx.at[0, kk]],
                            acc.at[pl.ds(kk * CH, CH)], sem)
           for kk in range(NCH)]
    for c in cps:
      c.wait()

    @pl.loop(1, L)
    def _(j):
      cps2 = [pltpu.async_copy(ingtab_hbm.at[iidx.at[j, kk]],
                               buf.at[pl.ds(kk * CH, CH)], sem)
              for kk in range(NCH)]
      for c in cps2:
        c.wait()

      @pl.loop(0, BPW, unroll=4)
      def _(i):
        plsc.addupdate(acc.at[i, pl.ds(0, 16)], buf[i, pl.ds(0, 16)])
        plsc.addupdate(acc.at[i, pl.ds(16, 16)], buf[i, pl.ds(16, 16)])

    pltpu.sync_copy(acc, ingo_hbm.at[pl.ds(base, BPW)])

  return k(rec_g, ing_g, id_table, ing_table)


def _mlp_body(id_ref, ing_ref, nut_ref, wn_ref, bn_ref, w1a_ref, w1b_ref,
              w1c_ref, b1_ref, w2_ref, b2_ref, out_ref):
  f32 = jnp.float32
  nut_feat = jnp.dot(nut_ref[...], wn_ref[...],
                     preferred_element_type=f32) + bn_ref[...]
  z = (jnp.dot(id_ref[...], w1a_ref[...], preferred_element_type=f32)
       + jnp.dot(ing_ref[...] * (1.0 / L), w1b_ref[...],
                 preferred_element_type=f32)
       + jnp.dot(nut_feat, w1c_ref[...], preferred_element_type=f32)
       + b1_ref[...])
  h = jnp.maximum(z, 0.0)
  out_ref[...] = jnp.dot(h, w2_ref[...], preferred_element_type=f32) + b2_ref[...]


def _tc_mlp(id_emb, ing_sum, nut_pad, wnT, bn, w1aT, w1bT, w1cT, b1, w2T, b2):
  BB = 2048
  grid = (B // BB,)
  full = lambda s: pl.BlockSpec(s, lambda i: (0, 0))
  return pl.pallas_call(
      _mlp_body,
      grid=grid,
      in_specs=[
          pl.BlockSpec((BB, D), lambda i: (i, 0)),
          pl.BlockSpec((BB, D), lambda i: (i, 0)),
          pl.BlockSpec((BB, 8), lambda i: (i, 0)),
          full((8, D)), full((1, D)), full((D, D)), full((D, D)),
          full((D, D)), full((1, D)), full((D, D)), full((1, D)),
      ],
      out_specs=pl.BlockSpec((BB, D), lambda i: (i, 0)),
      out_shape=jax.ShapeDtypeStruct((B, D), jnp.float32),
  )(id_emb, ing_sum, nut_pad, wnT, bn, w1aT, w1bT, w1cT, b1, w2T, b2)


def kernel(recipe_indices, ingredient_indices, nutrition_tensor, id_table,
           ing_table, W_nut, b_nut, W1, b1, W2, b2):
  rec_g = recipe_indices.astype(jnp.int32).reshape(NW, NCH, CH)
  ing_g = (ingredient_indices.astype(jnp.int32).T
           .reshape(L, NW, NCH, CH).transpose(1, 0, 2, 3))
  id_emb, ing_sum = _sc_gather(rec_g, ing_g, id_table, ing_table)

  nut_pad = jnp.pad(nutrition_tensor, ((0, 0), (0, 1)))
  wnT = jnp.pad(W_nut.T, ((0, 1), (0, 0)))
  out = _tc_mlp(
      id_emb, ing_sum, nut_pad, wnT, b_nut.reshape(1, D),
      W1[:, :D].T, W1[:, D:2 * D].T, W1[:, 2 * D:].T, b1.reshape(1, D),
      W2.T, b2.reshape(1, D))
  return out


# speedup vs baseline: 3.3399x; 3.3399x over previous
import functools

import jax
import jax.numpy as jnp
from jax import lax
from jax.experimental import pallas as pl
from jax.experimental.pallas import tpu as pltpu
from jax.experimental.pallas import tpu_sc as plsc

B = 16384
D = 32
L = 20
NC = 2
NS = 16
NW = NC * NS
BPW = B // NW
CH = 128
NCH = BPW // CH


def _sc_gather(rec_g, ing_g, id_table, ing_table):
  mesh = plsc.VectorSubcoreMesh(
      core_axis_name="c", subcore_axis_name="s",
      num_cores=NC, num_subcores=NS)

  @functools.partial(
      pl.kernel,
      out_type=(jax.ShapeDtypeStruct((B, D), jnp.float32),
                jax.ShapeDtypeStruct((B, D), jnp.float32)),
      mesh=mesh,
      scratch_types=[
          pltpu.VMEM((NCH, CH), jnp.int32),
          pltpu.VMEM((L, NCH, CH), jnp.int32),
          pltpu.VMEM((BPW, D), jnp.float32),
          pltpu.VMEM((BPW, D), jnp.float32),
          pltpu.VMEM((BPW, D), jnp.float32),
          pltpu.SemaphoreType.DMA,
      ],
      compiler_params=pltpu.CompilerParams(use_tc_tiling_on_sc=False),
  )
  def k(rec_hbm, ing_hbm, idtab_hbm, ingtab_hbm, ido_hbm, ingo_hbm,
        ridx, iidx, idrows, acc, buf, sem):
    wid = lax.axis_index("s") * NC + lax.axis_index("c")
    base = wid * BPW
    pltpu.sync_copy(rec_hbm.at[wid], ridx)
    pltpu.sync_copy(ing_hbm.at[wid], iidx)

    cps = [pltpu.async_copy(idtab_hbm.at[ridx.at[kk]],
                            idrows.at[pl.ds(kk * CH, CH)], sem)
           for kk in range(NCH)]
    for c in cps:
      c.wait()
    pltpu.sync_copy(idrows, ido_hbm.at[pl.ds(base, BPW)])

    cps = [pltpu.async_copy(ingtab_hbm.at[iidx.at[0, kk]],
                            acc.at[pl.ds(kk * CH, CH)], sem)
           for kk in range(NCH)]
    for c in cps:
      c.wait()

    @pl.loop(1, L)
    def _(j):
      cps2 = [pltpu.async_copy(ingtab_hbm.at[iidx.at[j, kk]],
                               buf.at[pl.ds(kk * CH, CH)], sem)
              for kk in range(NCH)]
      for c in cps2:
        c.wait()

      @pl.loop(0, BPW, unroll=4)
      def _(i):
        plsc.addupdate(acc.at[i, pl.ds(0, 16)], buf[i, pl.ds(0, 16)])
        plsc.addupdate(acc.at[i, pl.ds(16, 16)], buf[i, pl.ds(16, 16)])

    pltpu.sync_copy(acc, ingo_hbm.at[pl.ds(base, BPW)])

  return k(rec_g, ing_g, id_table, ing_table)


def _mlp_body(id_ref, ing_ref, nut_ref, wn_ref, bn_ref, w1a_ref, w1b_ref,
              w1c_ref, b1_ref, w2_ref, b2_ref, out_ref):
  f32 = jnp.float32
  nut_feat = jnp.dot(nut_ref[...], wn_ref[...],
                     preferred_element_type=f32) + bn_ref[...]
  z = (jnp.dot(id_ref[...], w1a_ref[...], preferred_element_type=f32)
       + jnp.dot(ing_ref[...] * (1.0 / L), w1b_ref[...],
                 preferred_element_type=f32)
       + jnp.dot(nut_feat, w1c_ref[...], preferred_element_type=f32)
       + b1_ref[...])
  h = jnp.maximum(z, 0.0)
  out_ref[...] = jnp.dot(h, w2_ref[...], preferred_element_type=f32) + b2_ref[...]


def _tc_mlp(id_emb, ing_sum, nut_pad, wnT, bn, w1aT, w1bT, w1cT, b1, w2T, b2):
  BB = 2048
  grid = (B // BB,)
  full = lambda s: pl.BlockSpec(s, lambda i: (0, 0))
  return pl.pallas_call(
      _mlp_body,
      grid=grid,
      in_specs=[
          pl.BlockSpec((BB, D), lambda i: (i, 0)),
          pl.BlockSpec((BB, D), lambda i: (i, 0)),
          pl.BlockSpec((BB, 8), lambda i: (i, 0)),
          full((8, D)), full((1, D)), full((D, D)), full((D, D)),
          full((D, D)), full((1, D)), full((D, D)), full((1, D)),
      ],
      out_specs=pl.BlockSpec((BB, D), lambda i: (i, 0)),
      out_shape=jax.ShapeDtypeStruct((B, D), jnp.float32),
  )(id_emb, ing_sum, nut_pad, wnT, bn, w1aT, w1bT, w1cT, b1, w2T, b2)


def kernel(recipe_indices, ingredient_indices, nutrition_tensor, id_table,
           ing_table, W_nut, b_nut, W1, b1, W2, b2):
  rec_g = recipe_indices.astype(jnp.int32).reshape(NW, NCH, CH)
  ing_g = (ingredient_indices.astype(jnp.int32).T
           .reshape(L, NW, NCH, CH).transpose(1, 0, 2, 3))
  id_emb, ing_sum = _sc_gather(rec_g, ing_g, id_table, ing_table)

  nut_pad = jnp.pad(nutrition_tensor, ((0, 0), (0, 1)))
  wnT = jnp.pad(W_nut.T, ((0, 1), (0, 0)))
  out = _tc_mlp(
      id_emb, ing_sum, nut_pad, wnT, b_nut.reshape(1, D),
      W1[:, :D].T, W1[:, D:2 * D].T, W1[:, 2 * D:].T, b1.reshape(1, D),
      W2.T, b2.reshape(1, D))
  return out
